# baseline (speedup 1.0000x reference)
import functools

import jax
import jax.numpy as jnp
from jax import lax
from jax.experimental import pallas as pl
from jax.experimental.pallas import tpu as pltpu
from jax.experimental.pallas import tpu_sc as plsc

CL = 4


def _keep_body(len_keep, nrow_ref, ncol_ref, kx_ref, mask_ref):
    L = nrow_ref.shape[-1]
    nvars = mask_ref.shape[-1]
    nj = nrow_ref[0]
    nl = ncol_ref[0]
    jidx = lax.broadcasted_iota(jnp.int32, (L, L), 1)
    lg = lax.broadcasted_iota(jnp.int32, (L, L), 0)
    cnt = (nj < nl) | ((nj == nl) & (jidx < lg))
    rank = jnp.sum(cnt.astype(jnp.int32), axis=1, keepdims=True)
    keep = (rank < len_keep).astype(jnp.float32)
    kx_ref[0] = jnp.broadcast_to(keep, (L, 128))
    mask_ref[0] = jnp.broadcast_to(1.0 - keep, (L, nvars))


def _sc_body(nvars, D, NS, xb_hbm, kx_hbm, out_hbm,
             vin, vout, kbuf, si0, si1, sk0, sk1, so0, so1):
    L = kx_hbm.shape[1]
    NCH = L // CL
    b = lax.axis_index("c") * NS + lax.axis_index("s")
    sems_in = (si0, si1)
    sems_k = (sk0, sk1)
    sems_out = (so0, so1)

    def in_data(c, rb):
        return pltpu.make_async_copy(
            xb_hbm.at[b, pl.ds(c * CL, CL)], vin.at[rb], sems_in[rb])

    def in_keep(c, rb):
        return pltpu.make_async_copy(
            kx_hbm.at[b, pl.ds(c * CL, CL)], kbuf.at[rb], sems_k[rb])

    def out_data(c, rb):
        return pltpu.make_async_copy(
            vout.at[rb], out_hbm.at[b, pl.ds(c * CL, CL)], sems_out[rb])

    def compute(rb):
        for l in range(CL):
            kv = kbuf[rb, l, pl.ds(0, 16)]
            for v in range(nvars):
                for k in range(D // 16):
                    sl = pl.ds(k * 16, 16)
                    vout[rb, l, v, sl] = vin[rb, l, v, sl] * kv

    in_data(0, 0).start()
    in_keep(0, 0).start()

    def body(i, carry):
        c0 = i * 2
        in_data(c0 + 1, 1).start()
        in_keep(c0 + 1, 1).start()

        @pl.when(c0 >= 2)
        def _():
            out_data(c0 - 2, 0).wait()

        in_data(c0, 0).wait()
        in_keep(c0, 0).wait()
        compute(0)
        out_data(c0, 0).start()

        @pl.when(c0 + 2 < NCH)
        def _():
            in_data(c0 + 2, 0).start()
            in_keep(c0 + 2, 0).start()

        @pl.when(c0 >= 2)
        def _():
            out_data(c0 - 1, 1).wait()

        in_data(c0 + 1, 1).wait()
        in_keep(c0 + 1, 1).wait()
        compute(1)
        out_data(c0 + 1, 1).start()
        return carry

    lax.fori_loop(0, NCH // 2, body, 0)
    out_data(NCH - 2, 0).wait()
    out_data(NCH - 1, 1).wait()


@jax.jit
def kernel(xb):
    bs, L, nvars, D = xb.shape
    len_keep = int(L * (1 - 0.15))
    noise = jax.random.uniform(jax.random.key(42), (bs, L), dtype=jnp.float32)
    nrow = noise.reshape(bs, 1, L)
    ncol = noise.reshape(bs, L, 1)

    kx, mask = pl.pallas_call(
        functools.partial(_keep_body, len_keep),
        grid=(bs,),
        in_specs=[
            pl.BlockSpec((1, 1, L), lambda b: (b, 0, 0)),
            pl.BlockSpec((1, L, 1), lambda b: (b, 0, 0)),
        ],
        out_specs=[
            pl.BlockSpec((1, L, 128), lambda b: (b, 0, 0)),
            pl.BlockSpec((1, L, nvars), lambda b: (b, 0, 0)),
        ],
        out_shape=[
            jax.ShapeDtypeStruct((bs, L, 128), jnp.float32),
            jax.ShapeDtypeStruct((bs, L, nvars), jnp.float32),
        ],
    )(nrow, ncol)

    NC, NS = 2, 16
    assert NC * NS == bs
    mesh = plsc.VectorSubcoreMesh(
        core_axis_name="c", subcore_axis_name="s", num_cores=NC)
    sc_fn = functools.partial(
        pl.kernel,
        mesh=mesh,
        compiler_params=pltpu.CompilerParams(use_tc_tiling_on_sc=True),
        out_type=jax.ShapeDtypeStruct((bs, L, nvars, D), jnp.float32),
        scratch_types=[
            pltpu.VMEM((2, CL, nvars, D), jnp.float32),
            pltpu.VMEM((2, CL, nvars, D), jnp.float32),
            pltpu.VMEM((2, CL, 128), jnp.float32),
            pltpu.SemaphoreType.DMA,
            pltpu.SemaphoreType.DMA,
            pltpu.SemaphoreType.DMA,
            pltpu.SemaphoreType.DMA,
            pltpu.SemaphoreType.DMA,
            pltpu.SemaphoreType.DMA,
        ],
    )(functools.partial(_sc_body, nvars, D, NS))
    x_masked = sc_fn(xb, kx)
    return x_masked, mask


# speedup vs baseline: 1.0638x; 1.0638x over previous
import functools

import jax
import jax.numpy as jnp
from jax import lax
from jax.experimental import pallas as pl
from jax.experimental.pallas import tpu as pltpu
from jax.experimental.pallas import tpu_sc as plsc

Q = 4
CH = 128


def _tc_body(len_keep, nrow_ref, ncol_ref, xb_hbm, out_hbm,
             sin, sout, in_sems, out_sems):
    L = nrow_ref.shape[-1]
    b = pl.program_id(0)
    nb = pl.num_programs(0)
    slot = lax.rem(b, 2)
    nslot = lax.rem(b + 1, 2)

    def in_copy(bi, s, q):
        return pltpu.make_async_copy(
            xb_hbm.at[bi, pl.ds(q * CH, CH)], sin.at[s, q], in_sems.at[s, q])

    def out_copy(bi, s, q):
        return pltpu.make_async_copy(
            sout.at[s, q], out_hbm.at[bi, pl.ds(q * CH, CH)], out_sems.at[s, q])

    @pl.when(b == 0)
    def _():
        for q in range(Q):
            in_copy(0, 0, q).start()

    @pl.when(b + 1 < nb)
    def _():
        for q in range(Q):
            in_copy(b + 1, nslot, q).start()

    @pl.when(b >= 2)
    def _():
        for q in range(Q):
            out_copy(b - 2, slot, q).wait()

    nj = nrow_ref[b]
    nl = ncol_ref[b]
    jidx = lax.broadcasted_iota(jnp.int32, (L, L), 1)
    lg = lax.broadcasted_iota(jnp.int32, (L, L), 0)
    cnt = (nj < nl) | ((nj == nl) & (jidx < lg))
    rank = jnp.sum(cnt.astype(jnp.int32), axis=1, keepdims=True)
    keep = (rank < len_keep).astype(jnp.float32)

    for q in range(Q):
        in_copy(b, slot, q).wait()
        kq = keep[q * CH:(q + 1) * CH]
        sout[slot, q] = sin[slot, q] * kq[:, :, None]
        out_copy(b, slot, q).start()

    @pl.when(b == nb - 1)
    def _():
        for q in range(Q):
            out_copy(b - 1, nslot, q).wait()
            out_copy(b, slot, q).wait()


def _sc_mask_body(nvars, NS, km_hbm, out_hbm, kmv, vbuf):
    L = km_hbm.shape[1]
    b = lax.axis_index("c") * NS + lax.axis_index("s")
    pltpu.sync_copy(km_hbm.at[b], kmv)
    for l in range(L):
        mv = kmv[l, pl.ds(0, 16)]
        vbuf[l, pl.ds(0, 16)] = mv
        vbuf[l, pl.ds(nvars - 16, 16)] = mv
    pltpu.sync_copy(vbuf, out_hbm.at[b])


@jax.jit
def kernel(xb):
    bs, L, nvars, D = xb.shape
    len_keep = int(L * (1 - 0.15))
    noise = jax.random.uniform(jax.random.key(42), (bs, L), dtype=jnp.float32)
    nrow = noise.reshape(bs, 1, L)
    ncol = noise.reshape(bs, L, 1)
    less = (noise[:, None, :] < noise[:, :, None])
    ties = ((noise[:, None, :] == noise[:, :, None])
            & (jnp.arange(L)[None, None, :] < jnp.arange(L)[None, :, None]))
    rank = (less | ties).sum(-1)
    keep = (rank < len_keep).astype(jnp.float32)
    km = (1.0 - keep)[:, :, None] + jnp.zeros((bs, L, 128), jnp.float32)

    NC, NS = 2, 16
    assert NC * NS == bs
    mesh = plsc.VectorSubcoreMesh(
        core_axis_name="c", subcore_axis_name="s", num_cores=NC)
    mask = pl.kernel(
        functools.partial(_sc_mask_body, nvars, NS),
        mesh=mesh,
        out_type=jax.ShapeDtypeStruct((bs, L, nvars), jnp.float32),
        scratch_types=[
            pltpu.VMEM((L, 128), jnp.float32),
            pltpu.VMEM((L, nvars), jnp.float32),
        ],
    )(km)

    x_masked, = pl.pallas_call(
        functools.partial(_tc_body, len_keep),
        grid=(bs,),
        in_specs=[
            pl.BlockSpec((bs, 1, L), lambda b: (0, 0, 0)),
            pl.BlockSpec((bs, L, 1), lambda b: (0, 0, 0)),
            pl.BlockSpec(memory_space=pl.ANY),
        ],
        out_specs=[
            pl.BlockSpec(memory_space=pl.ANY),
        ],
        out_shape=[
            jax.ShapeDtypeStruct((bs, L, nvars, D), xb.dtype),
        ],
        scratch_shapes=[
            pltpu.VMEM((2, Q, CH, nvars, D), jnp.float32),
            pltpu.VMEM((2, Q, CH, nvars, D), jnp.float32),
            pltpu.SemaphoreType.DMA((2, Q)),
            pltpu.SemaphoreType.DMA((2, Q)),
        ],
    )(nrow, ncol, xb)
    return x_masked, mask


# speedup vs baseline: 1.1470x; 1.0783x over previous
import functools

import jax
import jax.numpy as jnp
from jax import lax
from jax.experimental import pallas as pl
from jax.experimental.pallas import tpu as pltpu

Q = 4
CH = 128


def _body(len_keep, nrow_ref, ncol_ref, xb_hbm, out_hbm, mask_ref,
          sin, sout, in_sems, out_sems):
    L = nrow_ref.shape[-1]
    nvars = mask_ref.shape[-1]
    b = pl.program_id(0)
    nb = pl.num_programs(0)
    slot = lax.rem(b, 2)
    nslot = lax.rem(b + 1, 2)

    def in_copy(bi, s, q):
        return pltpu.make_async_copy(
            xb_hbm.at[bi, pl.ds(q * CH, CH)], sin.at[s, q], in_sems.at[s, q])

    def out_copy(bi, s, q):
        return pltpu.make_async_copy(
            sout.at[s, q], out_hbm.at[bi, pl.ds(q * CH, CH)], out_sems.at[s, q])

    @pl.when(b == 0)
    def _():
        for q in range(Q):
            in_copy(0, 0, q).start()

    @pl.when(b + 1 < nb)
    def _():
        for q in range(Q):
            in_copy(b + 1, nslot, q).start()

    @pl.when(b >= 2)
    def _():
        for q in range(Q):
            out_copy(b - 2, slot, q).wait()

    nj = nrow_ref[b]
    nl = ncol_ref[b]
    jidx = lax.broadcasted_iota(jnp.int32, (L, L), 1)
    lg = lax.broadcasted_iota(jnp.int32, (L, L), 0)
    cnt = (nj < nl) | ((nj == nl) & (jidx < lg))
    rank = jnp.sum(cnt.astype(jnp.int32), axis=1, keepdims=True)
    keep = (rank < len_keep).astype(jnp.float32)

    for q in range(Q):
        in_copy(b, slot, q).wait()
        kq = keep[q * CH:(q + 1) * CH]
        sout[slot, q] = sin[slot, q] * kq[:, :, None]
        out_copy(b, slot, q).start()

    mask_ref[0] = jnp.broadcast_to(1.0 - keep, (L, nvars))

    @pl.when(b == nb - 1)
    def _():
        for q in range(Q):
            out_copy(b - 1, nslot, q).wait()
            out_copy(b, slot, q).wait()


@jax.jit
def kernel(xb):
    bs, L, nvars, D = xb.shape
    len_keep = int(L * (1 - 0.15))
    noise = jax.random.uniform(jax.random.key(42), (bs, L), dtype=jnp.float32)
    nrow = noise.reshape(bs, 1, L)
    ncol = noise.reshape(bs, L, 1)

    x_masked, mask = pl.pallas_call(
        functools.partial(_body, len_keep),
        grid=(bs,),
        in_specs=[
            pl.BlockSpec((bs, 1, L), lambda b: (0, 0, 0)),
            pl.BlockSpec((bs, L, 1), lambda b: (0, 0, 0)),
            pl.BlockSpec(memory_space=pl.ANY),
        ],
        out_specs=[
            pl.BlockSpec(memory_space=pl.ANY),
            pl.BlockSpec((1, L, nvars), lambda b: (b, 0, 0)),
        ],
        out_shape=[
            jax.ShapeDtypeStruct((bs, L, nvars, D), xb.dtype),
            jax.ShapeDtypeStruct((bs, L, nvars), jnp.float32),
        ],
        scratch_shapes=[
            pltpu.VMEM((2, Q, CH, nvars, D), jnp.float32),
            pltpu.VMEM((2, Q, CH, nvars, D), jnp.float32),
            pltpu.SemaphoreType.DMA((2, Q)),
            pltpu.SemaphoreType.DMA((2, Q)),
        ],
    )(nrow, ncol, xb)
    return x_masked, mask
